# baseline (speedup 1.0000x reference)
import jax
import jax.numpy as jnp
from jax.experimental import pallas as pl
from jax.experimental.pallas import tpu as pltpu

_B, _R, _K = 64, 36, 16
_DQ, _DM, _MMF, _MMP, _MMC, _DC = 4800, 2048, 1200, 300, 32, 4


def _relu(x):
    return jnp.maximum(x, 0.0)


def _dot(a, b):
    return jnp.dot(a, b, preferred_element_type=jnp.float32)


def _fusion_kernel(q_ref, m_ref, wq_ref, bq_ref, wm_ref, bm_ref, wo_ref,
                   bo_ref, out_ref):
    a = _relu(_dot(q_ref[...], wq_ref[...]) + bq_ref[...])
    c = _relu(_dot(m_ref[...], wm_ref[...]) + bm_ref[...])
    out_ref[...] = _relu(_dot(a * c, wo_ref[...]) + bo_ref[...])


def _pair_kernel(m_ref, c_ref, wf0_ref, bf0_ref, wf1_ref, bf1_ref, wfo_ref,
                 bfo_ref, wc0_ref, bc0_ref, wc1_ref, bc1_ref, wco_ref,
                 bco_ref, out_ref):
    m = m_ref[0]
    c = c_ref[0]
    crit = jnp.sum(m * m, axis=1, keepdims=True)
    iota_r = jax.lax.broadcasted_iota(jnp.int32, (_R, 1), 0)
    o_row = jax.lax.broadcasted_iota(jnp.int32, (_K, _R), 0)
    o_col = jax.lax.broadcasted_iota(jnp.int32, (_K, _R), 1)
    ot_row = jax.lax.broadcasted_iota(jnp.int32, (_R, _K), 0)
    ot_col = jax.lax.broadcasted_iota(jnp.int32, (_R, _K), 1)
    O = jnp.zeros((_K, _R), jnp.float32)
    OT = jnp.zeros((_R, _K), jnp.float32)
    for t in range(_K):
        mx = jnp.max(crit)
        j = jnp.min(jnp.where(crit == mx, iota_r, _R))
        O = O + jnp.where((o_row == t) & (o_col == j), 1.0, 0.0)
        OT = OT + jnp.where((ot_row == j) & (ot_col == t), 1.0, 0.0)
        crit = jnp.where(iota_r == j, -1.0, crit)

    mt = _dot(O, m)
    ct = _dot(O, c)
    zi = _relu(_dot(mt, wf0_ref[...]) + bf0_ref[...])
    zj = _relu(_dot(mt, wf1_ref[...]) + bf1_ref[...])
    yi = _relu(_dot(ct, wc0_ref[...]) + bc0_ref[...])
    yj = _relu(_dot(ct, wc1_ref[...]) + bc1_ref[...])
    P = (zi[:, None, :] * zj[None, :, :]).reshape(_K * _K, _MMP)
    Pc = (yi[:, None, :] * yj[None, :, :]).reshape(_K * _K, _MMC)
    rij = _relu(_dot(P, wfo_ref[...]) + bfo_ref[...]) \
        + _relu(_dot(Pc, wco_ref[...]) + bco_ref[...])
    e = jnp.max(rij.reshape(_K, _K, _DM), axis=1)
    out_ref[0] = m + _dot(OT, e)


def kernel(q_expand, mm, coords,
           W_q0, b_q0, W_m1, b_m1, W_fo, b_fo,
           W_pf0, b_pf0, W_pf1, b_pf1, W_pfo, b_pfo,
           W_pc0, b_pc0, W_pc1, b_pc1, W_pco, b_pco):
    b, r, d = mm.shape
    mm_flat = mm.reshape(b * r, d)
    bm = 256
    grid1 = (b * r // bm,)

    mm_new = pl.pallas_call(
        _fusion_kernel,
        grid=grid1,
        in_specs=[
            pl.BlockSpec((bm, _DQ), lambda i: (i, 0)),
            pl.BlockSpec((bm, _DM), lambda i: (i, 0)),
            pl.BlockSpec((_DQ, _MMF), lambda i: (0, 0)),
            pl.BlockSpec((1, _MMF), lambda i: (0, 0)),
            pl.BlockSpec((_DM, _MMF), lambda i: (0, 0)),
            pl.BlockSpec((1, _MMF), lambda i: (0, 0)),
            pl.BlockSpec((_MMF, _DM), lambda i: (0, 0)),
            pl.BlockSpec((1, _DM), lambda i: (0, 0)),
        ],
        out_specs=pl.BlockSpec((bm, _DM), lambda i: (i, 0)),
        out_shape=jax.ShapeDtypeStruct((b * r, _DM), jnp.float32),
        compiler_params=pltpu.CompilerParams(
            dimension_semantics=("arbitrary",),
            vmem_limit_bytes=100 * 1024 * 1024),
    )(q_expand, mm_flat, W_q0, b_q0.reshape(1, -1), W_m1,
      b_m1.reshape(1, -1), W_fo, b_fo.reshape(1, -1))

    mm_new3 = mm_new.reshape(b, r, d)
    out = pl.pallas_call(
        _pair_kernel,
        grid=(b,),
        in_specs=[
            pl.BlockSpec((1, _R, _DM), lambda i: (i, 0, 0)),
            pl.BlockSpec((1, _R, _DC), lambda i: (i, 0, 0)),
            pl.BlockSpec((_DM, _MMP), lambda i: (0, 0)),
            pl.BlockSpec((1, _MMP), lambda i: (0, 0)),
            pl.BlockSpec((_DM, _MMP), lambda i: (0, 0)),
            pl.BlockSpec((1, _MMP), lambda i: (0, 0)),
            pl.BlockSpec((_MMP, _DM), lambda i: (0, 0)),
            pl.BlockSpec((1, _DM), lambda i: (0, 0)),
            pl.BlockSpec((_DC, _MMC), lambda i: (0, 0)),
            pl.BlockSpec((1, _MMC), lambda i: (0, 0)),
            pl.BlockSpec((_DC, _MMC), lambda i: (0, 0)),
            pl.BlockSpec((1, _MMC), lambda i: (0, 0)),
            pl.BlockSpec((_MMC, _DM), lambda i: (0, 0)),
            pl.BlockSpec((1, _DM), lambda i: (0, 0)),
        ],
        out_specs=pl.BlockSpec((1, _R, _DM), lambda i: (i, 0, 0)),
        out_shape=jax.ShapeDtypeStruct((b, r, d), jnp.float32),
        compiler_params=pltpu.CompilerParams(
            dimension_semantics=("arbitrary",)),
    )(mm_new3, coords, W_pf0, b_pf0.reshape(1, -1), W_pf1,
      b_pf1.reshape(1, -1), W_pfo, b_pfo.reshape(1, -1), W_pc0,
      b_pc0.reshape(1, -1), W_pc1, b_pc1.reshape(1, -1), W_pco,
      b_pco.reshape(1, -1))
    return out


# speedup vs baseline: 1.4293x; 1.4293x over previous
import jax
import jax.numpy as jnp
from jax.experimental import pallas as pl
from jax.experimental.pallas import tpu as pltpu

_B, _R, _K = 64, 36, 16
_DQ, _DM, _MMF, _MMP, _MMC, _DC = 4800, 2048, 1200, 300, 32, 4


def _relu(x):
    return jnp.maximum(x, 0.0)


def _dot(a, b):
    return jnp.dot(a, b, preferred_element_type=jnp.float32)


def _fusion_kernel(q_ref, m_ref, wq_ref, bq_ref, wm_ref, bm_ref, wo_ref,
                   bo_ref, out_ref):
    a = _relu(_dot(q_ref[...], wq_ref[...]) + bq_ref[...])
    c = _relu(_dot(m_ref[...], wm_ref[...]) + bm_ref[...])
    out_ref[...] = _relu(_dot(a * c, wo_ref[...]) + bo_ref[...])


_G = 8


def _img_of(iota):
    acc = jnp.zeros(iota.shape, jnp.int32)
    for g in range(1, _G):
        acc = acc + (iota >= g * _R).astype(jnp.int32)
    return acc


def _pair_kernel(m_ref, c_ref, wf0_ref, bf0_ref, wf1_ref, bf1_ref, wfo_ref,
                 bfo_ref, wc0_ref, bc0_ref, wc1_ref, bc1_ref, wco_ref,
                 bco_ref, out_ref):
    gr = _G * _R
    gk = _G * _K
    m = m_ref[...]
    c = c_ref[...]
    crit = jnp.sum(m * m, axis=1, keepdims=True)

    row_i = jax.lax.broadcasted_iota(jnp.int32, (gr, gr), 0)
    col_i = jax.lax.broadcasted_iota(jnp.int32, (gr, gr), 1)
    same = _img_of(row_i) == _img_of(col_i)
    cb = jnp.transpose(crit)
    ahead = same & ((cb > crit) | ((cb == crit) & (col_i < row_i)))
    rank = jnp.sum(ahead.astype(jnp.int32), axis=1, keepdims=True)

    ot_r = jax.lax.broadcasted_iota(jnp.int32, (gr, gk), 0)
    ot_c = jax.lax.broadcasted_iota(jnp.int32, (gr, gk), 1)
    OT = ((_img_of(ot_r) == (ot_c >> 4)) & (rank == (ot_c & 15))
          ).astype(jnp.float32)

    gather_dn = (((0,), (0,)), ((), ()))
    mt = jax.lax.dot_general(OT, m, gather_dn,
                             preferred_element_type=jnp.float32)
    ct = jax.lax.dot_general(OT, c, gather_dn,
                             preferred_element_type=jnp.float32)

    zi = _relu(_dot(mt, wf0_ref[...]) + bf0_ref[...])
    zj = _relu(_dot(mt, wf1_ref[...]) + bf1_ref[...])
    yi = _relu(_dot(ct, wc0_ref[...]) + bc0_ref[...])
    yj = _relu(_dot(ct, wc1_ref[...]) + bc1_ref[...])

    def pair(a, width):
        ai = jnp.broadcast_to(a[:, None, :], (gk, _K, width)) \
               .reshape(gk * _K, width)
        a4 = a.reshape(_G, _K, width)
        aj = jnp.broadcast_to(a4[:, None, :, :], (_G, _K, _K, width)) \
               .reshape(gk * _K, width)
        return ai, aj

    P = pair(zi, _MMP)[0] * pair(zj, _MMP)[1]
    Pc = pair(yi, _MMC)[0] * pair(yj, _MMC)[1]
    rij = _relu(_dot(P, wfo_ref[...]) + bfo_ref[...]) \
        + _relu(_dot(Pc, wco_ref[...]) + bco_ref[...])
    e = jnp.max(rij.reshape(gk, _K, _DM), axis=1)
    out_ref[...] = m + _dot(OT, e)


def kernel(q_expand, mm, coords,
           W_q0, b_q0, W_m1, b_m1, W_fo, b_fo,
           W_pf0, b_pf0, W_pf1, b_pf1, W_pfo, b_pfo,
           W_pc0, b_pc0, W_pc1, b_pc1, W_pco, b_pco):
    b, r, d = mm.shape
    mm_flat = mm.reshape(b * r, d)
    bm = 256
    grid1 = (b * r // bm,)

    mm_new = pl.pallas_call(
        _fusion_kernel,
        grid=grid1,
        in_specs=[
            pl.BlockSpec((bm, _DQ), lambda i: (i, 0)),
            pl.BlockSpec((bm, _DM), lambda i: (i, 0)),
            pl.BlockSpec((_DQ, _MMF), lambda i: (0, 0)),
            pl.BlockSpec((1, _MMF), lambda i: (0, 0)),
            pl.BlockSpec((_DM, _MMF), lambda i: (0, 0)),
            pl.BlockSpec((1, _MMF), lambda i: (0, 0)),
            pl.BlockSpec((_MMF, _DM), lambda i: (0, 0)),
            pl.BlockSpec((1, _DM), lambda i: (0, 0)),
        ],
        out_specs=pl.BlockSpec((bm, _DM), lambda i: (i, 0)),
        out_shape=jax.ShapeDtypeStruct((b * r, _DM), jnp.float32),
        compiler_params=pltpu.CompilerParams(
            dimension_semantics=("arbitrary",),
            vmem_limit_bytes=100 * 1024 * 1024),
    )(q_expand, mm_flat, W_q0, b_q0.reshape(1, -1), W_m1,
      b_m1.reshape(1, -1), W_fo, b_fo.reshape(1, -1))

    coords_flat = coords.reshape(b * r, _DC)
    out = pl.pallas_call(
        _pair_kernel,
        grid=(b // _G,),
        in_specs=[
            pl.BlockSpec((_G * _R, _DM), lambda i: (i, 0)),
            pl.BlockSpec((_G * _R, _DC), lambda i: (i, 0)),
            pl.BlockSpec((_DM, _MMP), lambda i: (0, 0)),
            pl.BlockSpec((1, _MMP), lambda i: (0, 0)),
            pl.BlockSpec((_DM, _MMP), lambda i: (0, 0)),
            pl.BlockSpec((1, _MMP), lambda i: (0, 0)),
            pl.BlockSpec((_MMP, _DM), lambda i: (0, 0)),
            pl.BlockSpec((1, _DM), lambda i: (0, 0)),
            pl.BlockSpec((_DC, _MMC), lambda i: (0, 0)),
            pl.BlockSpec((1, _MMC), lambda i: (0, 0)),
            pl.BlockSpec((_DC, _MMC), lambda i: (0, 0)),
            pl.BlockSpec((1, _MMC), lambda i: (0, 0)),
            pl.BlockSpec((_MMC, _DM), lambda i: (0, 0)),
            pl.BlockSpec((1, _DM), lambda i: (0, 0)),
        ],
        out_specs=pl.BlockSpec((_G * _R, _DM), lambda i: (i, 0)),
        out_shape=jax.ShapeDtypeStruct((b * r, d), jnp.float32),
        compiler_params=pltpu.CompilerParams(
            dimension_semantics=("arbitrary",),
            vmem_limit_bytes=100 * 1024 * 1024),
    )(mm_new, coords_flat, W_pf0, b_pf0.reshape(1, -1), W_pf1,
      b_pf1.reshape(1, -1), W_pfo, b_pfo.reshape(1, -1), W_pc0,
      b_pc0.reshape(1, -1), W_pc1, b_pc1.reshape(1, -1), W_pco,
      b_pco.reshape(1, -1))
    return out.reshape(b, r, d)


# speedup vs baseline: 1.4476x; 1.0128x over previous
import jax
import jax.numpy as jnp
from jax.experimental import pallas as pl
from jax.experimental.pallas import tpu as pltpu

_B, _R, _K = 64, 36, 16
_DQ, _DM, _MMF, _MMP, _MMC, _DC = 4800, 2048, 1200, 300, 32, 4
_G = 8
_H = 2


def _relu(x):
    return jnp.maximum(x, 0.0)


def _dot(a, b):
    return jnp.dot(a, b, preferred_element_type=jnp.float32)


def _img_of(iota):
    acc = jnp.zeros(iota.shape, jnp.int32)
    for g in range(1, _G):
        acc = acc + (iota >= g * _R).astype(jnp.int32)
    return acc


def _qbranch_kernel(q_ref, wq_ref, bq_ref, out_ref):
    out_ref[...] = _relu(_dot(q_ref[...], wq_ref[...]) + bq_ref[...])


def _murel_kernel(a_ref, m_ref, c_ref,
                  wm_ref, bm_ref, wo_ref, bo_ref,
                  wf0_ref, bf0_ref, wf1_ref, bf1_ref, wfo_ref, bfo_ref,
                  wc0_ref, bc0_ref, wc1_ref, bc1_ref, wco_ref, bco_ref,
                  out_ref):
    gr = _G * _R
    gk = _G * _K

    g = _relu(_dot(m_ref[...], wm_ref[...]) + bm_ref[...])
    m = _relu(_dot(a_ref[...] * g, wo_ref[...]) + bo_ref[...])

    c = c_ref[...]
    crit = jnp.sum(m * m, axis=1, keepdims=True)

    row_i = jax.lax.broadcasted_iota(jnp.int32, (gr, gr), 0)
    col_i = jax.lax.broadcasted_iota(jnp.int32, (gr, gr), 1)
    same = _img_of(row_i) == _img_of(col_i)
    cb = jnp.transpose(crit)
    ahead = same & ((cb > crit) | ((cb == crit) & (col_i < row_i)))
    rank = jnp.sum(ahead.astype(jnp.int32), axis=1, keepdims=True)

    ot_r = jax.lax.broadcasted_iota(jnp.int32, (gr, gk), 0)
    ot_c = jax.lax.broadcasted_iota(jnp.int32, (gr, gk), 1)
    OT = ((_img_of(ot_r) == (ot_c >> 4)) & (rank == (ot_c & 15))
          ).astype(jnp.float32)

    gather_dn = (((0,), (0,)), ((), ()))
    mt = jax.lax.dot_general(OT, m, gather_dn,
                             preferred_element_type=jnp.float32)
    ct = jax.lax.dot_general(OT, c, gather_dn,
                             preferred_element_type=jnp.float32)

    zi = _relu(_dot(mt, wf0_ref[...]) + bf0_ref[...])
    zj = _relu(_dot(mt, wf1_ref[...]) + bf1_ref[...])
    yi = _relu(_dot(ct, wc0_ref[...]) + bc0_ref[...])
    yj = _relu(_dot(ct, wc1_ref[...]) + bc1_ref[...])

    gh = _G // _H
    hk = gh * _K

    def pair(ai, aj, width):
        pi = jnp.broadcast_to(ai[:, None, :], (hk, _K, width)) \
               .reshape(hk * _K, width)
        a4 = aj.reshape(gh, _K, width)
        pj = jnp.broadcast_to(a4[:, None, :, :], (gh, _K, _K, width)) \
               .reshape(hk * _K, width)
        return pi * pj

    es = []
    for h in range(_H):
        sl = slice(h * hk, (h + 1) * hk)
        P = pair(zi[sl], zj[sl], _MMP)
        Pc = pair(yi[sl], yj[sl], _MMC)
        rij = _relu(_dot(P, wfo_ref[...]) + bfo_ref[...]) \
            + _relu(_dot(Pc, wco_ref[...]) + bco_ref[...])
        es.append(jnp.max(rij.reshape(hk, _K, _DM), axis=1))
    e = jnp.concatenate(es, axis=0)
    out_ref[...] = m + _dot(OT, e)


def kernel(q_expand, mm, coords,
           W_q0, b_q0, W_m1, b_m1, W_fo, b_fo,
           W_pf0, b_pf0, W_pf1, b_pf1, W_pfo, b_pfo,
           W_pc0, b_pc0, W_pc1, b_pc1, W_pco, b_pco):
    b, r, d = mm.shape
    gr = _G * _R
    mm_flat = mm.reshape(b * r, d)
    coords_flat = coords.reshape(b * r, _DC)

    def const(shape):
        nd = len(shape)
        return pl.BlockSpec(shape, lambda i: (0,) * nd)

    bm = 256
    a_all = pl.pallas_call(
        _qbranch_kernel,
        grid=(b * r // bm,),
        in_specs=[
            pl.BlockSpec((bm, _DQ), lambda i: (i, 0)),
            const((_DQ, _MMF)), const((1, _MMF)),
        ],
        out_specs=pl.BlockSpec((bm, _MMF), lambda i: (i, 0)),
        out_shape=jax.ShapeDtypeStruct((b * r, _MMF), jnp.float32),
        compiler_params=pltpu.CompilerParams(
            dimension_semantics=("arbitrary",),
            vmem_limit_bytes=60 * 1024 * 1024),
    )(q_expand, W_q0, b_q0.reshape(1, -1))

    out = pl.pallas_call(
        _murel_kernel,
        grid=(b // _G,),
        in_specs=[
            pl.BlockSpec((gr, _MMF), lambda i: (i, 0)),
            pl.BlockSpec((gr, _DM), lambda i: (i, 0)),
            pl.BlockSpec((gr, _DC), lambda i: (i, 0)),
            const((_DM, _MMF)), const((1, _MMF)),
            const((_MMF, _DM)), const((1, _DM)),
            const((_DM, _MMP)), const((1, _MMP)),
            const((_DM, _MMP)), const((1, _MMP)),
            const((_MMP, _DM)), const((1, _DM)),
            const((_DC, _MMC)), const((1, _MMC)),
            const((_DC, _MMC)), const((1, _MMC)),
            const((_MMC, _DM)), const((1, _DM)),
        ],
        out_specs=pl.BlockSpec((gr, _DM), lambda i: (i, 0)),
        out_shape=jax.ShapeDtypeStruct((b * r, d), jnp.float32),
        compiler_params=pltpu.CompilerParams(
            dimension_semantics=("arbitrary",),
            vmem_limit_bytes=62 * 1024 * 1024),
    )(a_all, mm_flat, coords_flat,
      W_m1, b_m1.reshape(1, -1), W_fo, b_fo.reshape(1, -1),
      W_pf0, b_pf0.reshape(1, -1), W_pf1, b_pf1.reshape(1, -1),
      W_pfo, b_pfo.reshape(1, -1),
      W_pc0, b_pc0.reshape(1, -1), W_pc1, b_pc1.reshape(1, -1),
      W_pco, b_pco.reshape(1, -1))
    return out.reshape(b, r, d)
